# Initial kernel scaffold; baseline (speedup 1.0000x reference)
import jax
import jax.numpy as jnp
from jax.experimental import pallas as pl


def kernel(x):
    raise NotImplementedError("write your pallas kernel here")


# speedup vs baseline: 14.1300x; 14.1300x over previous
import dataclasses

import jax
import jax.numpy as jnp
from jax import lax
from jax.experimental import pallas as pl
from jax.experimental.pallas import tpu as pltpu
from jax.experimental.pallas import tpu_sc as plsc

_CP = pltpu.CompilerParams()
if "needs_layout_passes" in pltpu.CompilerParams.__dataclass_fields__:
    _CP = dataclasses.replace(_CP, needs_layout_passes=False)

D = 8
ROWS = 128
T = 262144
CH = 32768
N_CHUNKS = T // CH
OUT_CH = CH // D
LANES = 16


def _sc_downsample(xr):
    mesh = plsc.VectorSubcoreMesh(core_axis_name="core",
                                  subcore_axis_name="subcore")

    @pl.kernel(out_type=jax.ShapeDtypeStruct((ROWS, T // D), jnp.float32),
               mesh=mesh, compiler_params=_CP)
    def k(x_hbm, o_hbm):
        def body(in_vmem, out_vmem):
            base = lax.iota(jnp.int32, LANES) * D
            zero = jnp.zeros((LANES,), jnp.int32)

            @pl.loop(0, OUT_CH // LANES)
            def _(j):
                idx = base + j * (D * LANES)
                vals = plsc.load_gather(in_vmem, [zero, idx])
                out_vmem[0, pl.ds(j * LANES, LANES)] = vals

        pltpu.emit_pipeline(
            body,
            grid=(ROWS, N_CHUNKS),
            in_specs=[pl.BlockSpec((1, CH), lambda i, j: (i, j))],
            out_specs=[pl.BlockSpec((1, OUT_CH), lambda i, j: (i, j))],
            core_axis_name=("core", "subcore"),
            dimension_semantics=(pltpu.PARALLEL, pltpu.PARALLEL),
        )(x_hbm, o_hbm)

    return k(xr)


def kernel(x):
    b, c, p, t = x.shape
    xr = x.reshape(ROWS, T)
    out = _sc_downsample(xr)
    return out.reshape(b, c, p, T // D)


# speedup vs baseline: 14.1330x; 1.0002x over previous
import dataclasses

import jax
import jax.numpy as jnp
from jax import lax
from jax.experimental import pallas as pl
from jax.experimental.pallas import tpu as pltpu
from jax.experimental.pallas import tpu_sc as plsc

_CP = pltpu.CompilerParams()
if "needs_layout_passes" in pltpu.CompilerParams.__dataclass_fields__:
    _CP = dataclasses.replace(_CP, needs_layout_passes=False)

D = 8
ROWS = 128
T = 262144
CH = 32768
N_CHUNKS = T // CH
OUT_CH = CH // D
LANES = 16


def _sc_downsample(xr):
    mesh = plsc.VectorSubcoreMesh(core_axis_name="core",
                                  subcore_axis_name="subcore")

    @pl.kernel(out_type=jax.ShapeDtypeStruct((ROWS, T // D), jnp.float32),
               mesh=mesh, compiler_params=_CP)
    def k(x_hbm, o_hbm):
        def body(in_vmem, out_vmem):
            base = lax.iota(jnp.int32, LANES) * D
            zero = jnp.zeros((LANES,), jnp.int32)

            @plsc.parallel_loop(0, OUT_CH // LANES, unroll=8)
            def _(j):
                idx = base + j * (D * LANES)
                vals = plsc.load_gather(in_vmem, [zero, idx])
                out_vmem[0, pl.ds(j * LANES, LANES)] = vals

        pltpu.emit_pipeline(
            body,
            grid=(ROWS, N_CHUNKS),
            in_specs=[pl.BlockSpec((1, CH), lambda i, j: (i, j))],
            out_specs=[pl.BlockSpec((1, OUT_CH), lambda i, j: (i, j))],
            core_axis_name=("core", "subcore"),
            dimension_semantics=(pltpu.PARALLEL, pltpu.PARALLEL),
        )(x_hbm, o_hbm)

    return k(xr)


def kernel(x):
    b, c, p, t = x.shape
    xr = x.reshape(ROWS, T)
    out = _sc_downsample(xr)
    return out.reshape(b, c, p, T // D)


# speedup vs baseline: 32.6036x; 2.3069x over previous
import dataclasses

import jax
import jax.numpy as jnp
from jax import lax
from jax.experimental import pallas as pl
from jax.experimental.pallas import tpu as pltpu
from jax.experimental.pallas import tpu_sc as plsc

_CP = pltpu.CompilerParams()
for _f, _v in (("needs_layout_passes", False), ("use_tc_tiling_on_sc", True)):
    if _f in pltpu.CompilerParams.__dataclass_fields__:
        _CP = dataclasses.replace(_CP, **{_f: _v})

D = 8
B, C, P = 16, 4, 2
T = 262144
CH = 16384
N_CHUNKS = T // CH
OUT_CH = CH // D
LANES = 16


def _sc_downsample(x):
    mesh = plsc.VectorSubcoreMesh(core_axis_name="core",
                                  subcore_axis_name="subcore")

    @pl.kernel(out_type=jax.ShapeDtypeStruct((B, C, P, T // D), jnp.float32),
               mesh=mesh, compiler_params=_CP)
    def k(x_hbm, o_hbm):
        def body(in_vmem, out_vmem):
            base = lax.iota(jnp.int32, LANES) * D
            zero = jnp.zeros((LANES,), jnp.int32)

            for p in range(P):
                pvec = jnp.full((LANES,), p, jnp.int32)

                @pl.loop(0, OUT_CH // LANES)
                def _(j, pvec=pvec, p=p):
                    idx = base + j * (D * LANES)
                    vals = plsc.load_gather(in_vmem, [zero, zero, pvec, idx])
                    out_vmem[0, 0, p, pl.ds(j * LANES, LANES)] = vals

        pltpu.emit_pipeline(
            body,
            grid=(B, C, N_CHUNKS),
            in_specs=[pl.BlockSpec((1, 1, P, CH), lambda i, j, k: (i, j, 0, k))],
            out_specs=[pl.BlockSpec((1, 1, P, OUT_CH),
                                    lambda i, j, k: (i, j, 0, k))],
            core_axis_name=("core", "subcore"),
            dimension_semantics=(pltpu.PARALLEL, pltpu.PARALLEL, pltpu.PARALLEL),
        )(x_hbm, o_hbm)

    return k(x)


def kernel(x):
    return _sc_downsample(x)


# speedup vs baseline: 52.8237x; 1.6202x over previous
import dataclasses

import jax
import jax.numpy as jnp
from jax import lax
from jax.experimental import pallas as pl
from jax.experimental.pallas import tpu as pltpu
from jax.experimental.pallas import tpu_sc as plsc

_CP = pltpu.CompilerParams()
for _f, _v in (("needs_layout_passes", False), ("use_tc_tiling_on_sc", True)):
    if _f in pltpu.CompilerParams.__dataclass_fields__:
        _CP = dataclasses.replace(_CP, **{_f: _v})

D = 8
B, C, P = 16, 4, 2
T = 262144
CH = 16384
N_CHUNKS = T // CH
OUT_CH = CH // D
LANES = 16


def _sc_downsample(x):
    mesh = plsc.VectorSubcoreMesh(core_axis_name="core",
                                  subcore_axis_name="subcore")

    @pl.kernel(out_type=jax.ShapeDtypeStruct((B, C, P, T // D), jnp.float32),
               mesh=mesh, compiler_params=_CP)
    def k(x_hbm, o_hbm):
        def body(in_vmem, out_vmem):
            base = lax.iota(jnp.int32, LANES) * D
            zero = jnp.zeros((LANES,), jnp.int32)

            for p in range(P):
                pvec = jnp.full((LANES,), p, jnp.int32)

                @pl.loop(0, OUT_CH // LANES)
                def _(j, pvec=pvec, p=p):
                    idx = base + j * (D * LANES)
                    vals = plsc.load_gather(in_vmem, [zero, zero, pvec, idx])
                    out_vmem[0, 0, p, pl.ds(j * LANES, LANES)] = vals

        pltpu.emit_pipeline(
            body,
            grid=(B * C, N_CHUNKS),
            in_specs=[pl.BlockSpec((1, 1, P, CH),
                                   lambda f, k: (f // C, f % C, 0, k))],
            out_specs=[pl.BlockSpec((1, 1, P, OUT_CH),
                                    lambda f, k: (f // C, f % C, 0, k))],
            core_axis_name=("core", "subcore"),
            dimension_semantics=(pltpu.PARALLEL, pltpu.PARALLEL),
        )(x_hbm, o_hbm)

    return k(x)


def kernel(x):
    return _sc_downsample(x)
